# baseline (speedup 1.0000x reference)
import functools

import jax
import jax.numpy as jnp
from jax import lax
from jax.experimental import pallas as pl
from jax.experimental.pallas import tpu as pltpu
from jax.experimental.pallas import tpu_sc as plsc

B = 16384
F = 100
D = 32
G = 20
S = 5
L = 16

NC = 2
NS = 16
NW = NC * NS

GSC = 8
GTC = G - GSC
CW = 2048

W = 256
NSTRIPE = B // W
NUNIT = NSTRIPE * GSC // NW
IR = S * D
OR = D


def _body(in_hbm, out_hbm, iv0, iv1, ov0, ov1, si0, si1, so0, so1):
    wid = lax.axis_index("s") * NC + lax.axis_index("c")
    in_bufs = (iv0, iv1)
    out_bufs = (ov0, ov1)
    in_sems = (si0, si1)
    out_sems = (so0, so1)

    def unit_addr(u):
        ug = wid * NUNIT + u
        stripe = ug // GSC
        g = ug % GSC
        return g, stripe * W

    def start_in(u, b):
        g, col = unit_addr(u)
        pltpu.make_async_copy(
            in_hbm.at[pl.ds(IR * g, IR), pl.ds(col, W)], in_bufs[b], in_sems[b]
        ).start()

    def wait_in(b):
        pltpu.make_async_copy(
            in_hbm.at[pl.ds(0, IR), pl.ds(0, W)], in_bufs[b], in_sems[b]
        ).wait()

    def start_out(u, b):
        g, col = unit_addr(u)
        pltpu.make_async_copy(
            out_bufs[b], out_hbm.at[pl.ds(OR * g, OR), pl.ds(col, W)], out_sems[b]
        ).start()

    def wait_out(b):
        pltpu.make_async_copy(
            out_bufs[b], out_hbm.at[pl.ds(0, OR), pl.ds(0, W)], out_sems[b]
        ).wait()

    def compute(b):
        in_v = in_bufs[b]
        out_v = out_bufs[b]

        def row_body(d, carry):
            for v in range(W // L):
                acc = in_v[d, pl.ds(v * L, L)]
                for j in range(1, S):
                    acc = acc + in_v[j * D + d, pl.ds(v * L, L)]
                out_v[d, pl.ds(v * L, L)] = acc * (1.0 / S)
            return carry

        lax.fori_loop(0, OR, row_body, 0)

    start_in(0, 0)
    start_in(1, 1)

    def loop_body(i, carry):
        for b in range(2):
            u = 2 * i + b
            wait_in(b)

            @pl.when(u >= 2)
            def _():
                wait_out(b)

            compute(b)
            start_out(u, b)

            @pl.when(u + 2 < NUNIT)
            def _():
                start_in(u + 2, b)

        return carry

    lax.fori_loop(0, NUNIT // 2, loop_body, 0)
    wait_out(0)
    wait_out(1)


def _tc_body(x_ref, o_ref):
    x = x_ref[...]
    acc = x[0:D]
    for j in range(1, S):
        acc = acc + x[j * D:(j + 1) * D]
    o_ref[...] = acc * (1.0 / S)


@jax.jit
def kernel(emb_vector):
    x_t = jnp.transpose(emb_vector, (1, 2, 0)).reshape(F * D, B)
    mesh = plsc.VectorSubcoreMesh(core_axis_name="c", subcore_axis_name="s")
    sc_out = pl.kernel(
        _body,
        out_type=jax.ShapeDtypeStruct((GSC * D, B), jnp.float32),
        mesh=mesh,
        compiler_params=pltpu.CompilerParams(use_tc_tiling_on_sc=True),
        scratch_types=[
            pltpu.VMEM((IR, W), jnp.float32),
            pltpu.VMEM((IR, W), jnp.float32),
            pltpu.VMEM((OR, W), jnp.float32),
            pltpu.VMEM((OR, W), jnp.float32),
            pltpu.SemaphoreType.DMA,
            pltpu.SemaphoreType.DMA,
            pltpu.SemaphoreType.DMA,
            pltpu.SemaphoreType.DMA,
        ],
    )(x_t)
    tc_out = pl.pallas_call(
        _tc_body,
        grid=(GTC, B // CW),
        in_specs=[pl.BlockSpec((IR, CW), lambda g, c: (GSC + g, c))],
        out_specs=pl.BlockSpec((OR, CW), lambda g, c: (GSC + g, c)),
        out_shape=jax.ShapeDtypeStruct((G * D, B), jnp.float32),
    )(x_t)
    out_t = lax.dynamic_update_slice(tc_out, sc_out, (0, 0))
    return jnp.transpose(out_t.reshape(G, D, B), (2, 0, 1))


# speedup vs baseline: 1.1689x; 1.1689x over previous
import functools

import jax
import jax.numpy as jnp
from jax import lax
from jax.experimental import pallas as pl
from jax.experimental.pallas import tpu as pltpu
from jax.experimental.pallas import tpu_sc as plsc

B = 16384
F = 100
D = 32
G = 20
S = 5
L = 16

NC = 2
NS = 16
NW = NC * NS

W = 256
NSTRIPE = B // W
NUNIT = NSTRIPE * G // NW
IR = S * D
OR = D


def _body(in_hbm, out_hbm, iv0, iv1, ov0, ov1, si0, si1, so0, so1):
    wid = lax.axis_index("s") * NC + lax.axis_index("c")
    in_bufs = (iv0, iv1)
    out_bufs = (ov0, ov1)
    in_sems = (si0, si1)
    out_sems = (so0, so1)

    def unit_addr(u):
        ug = wid * NUNIT + u
        stripe = ug // G
        g = ug % G
        return g, stripe * W

    def start_in(u, b):
        g, col = unit_addr(u)
        pltpu.make_async_copy(
            in_hbm.at[pl.ds(IR * g, IR), pl.ds(col, W)], in_bufs[b], in_sems[b]
        ).start()

    def wait_in(b):
        pltpu.make_async_copy(
            in_hbm.at[pl.ds(0, IR), pl.ds(0, W)], in_bufs[b], in_sems[b]
        ).wait()

    def start_out(u, b):
        g, col = unit_addr(u)
        pltpu.make_async_copy(
            out_bufs[b], out_hbm.at[pl.ds(OR * g, OR), pl.ds(col, W)], out_sems[b]
        ).start()

    def wait_out(b):
        pltpu.make_async_copy(
            out_bufs[b], out_hbm.at[pl.ds(0, OR), pl.ds(0, W)], out_sems[b]
        ).wait()

    def compute(b):
        in_v = in_bufs[b]
        out_v = out_bufs[b]

        def row_body(d, carry):
            for v in range(W // L):
                acc = in_v[d, pl.ds(v * L, L)]
                for j in range(1, S):
                    acc = acc + in_v[j * D + d, pl.ds(v * L, L)]
                out_v[d, pl.ds(v * L, L)] = acc * (1.0 / S)
            return carry

        lax.fori_loop(0, OR, row_body, 0)

    start_in(0, 0)
    start_in(1, 1)

    def loop_body(i, carry):
        for b in range(2):
            u = 2 * i + b
            wait_in(b)

            @pl.when(u >= 2)
            def _():
                wait_out(b)

            compute(b)
            start_out(u, b)

            @pl.when(u + 2 < NUNIT)
            def _():
                start_in(u + 2, b)

        return carry

    lax.fori_loop(0, NUNIT // 2, loop_body, 0)
    wait_out(0)
    wait_out(1)


@jax.jit
def kernel(emb_vector):
    x_t = jnp.transpose(emb_vector, (1, 2, 0)).reshape(F * D, B)
    mesh = plsc.VectorSubcoreMesh(core_axis_name="c", subcore_axis_name="s")
    out_t = pl.kernel(
        _body,
        out_type=jax.ShapeDtypeStruct((G * D, B), jnp.float32),
        mesh=mesh,
        compiler_params=pltpu.CompilerParams(use_tc_tiling_on_sc=True),
        scratch_types=[
            pltpu.VMEM((IR, W), jnp.float32),
            pltpu.VMEM((IR, W), jnp.float32),
            pltpu.VMEM((OR, W), jnp.float32),
            pltpu.VMEM((OR, W), jnp.float32),
            pltpu.SemaphoreType.DMA,
            pltpu.SemaphoreType.DMA,
            pltpu.SemaphoreType.DMA,
            pltpu.SemaphoreType.DMA,
        ],
    )(x_t)
    return jnp.transpose(out_t.reshape(G, D, B), (2, 0, 1))


# speedup vs baseline: 1.1995x; 1.0262x over previous
import functools

import jax
import jax.numpy as jnp
from jax import lax
from jax.experimental import pallas as pl
from jax.experimental.pallas import tpu as pltpu
from jax.experimental.pallas import tpu_sc as plsc

B = 16384
F = 100
D = 32
G = 20
S = 5
L = 16

NC = 2
NS = 16
NW = NC * NS

W = 256
NSTRIPE = B // W
NUNIT = NSTRIPE * G // NW
IR = S * D
OR = D


def _body(in_hbm, out_hbm, iv0, iv1, ov0, ov1, si0, si1, so0, so1):
    wid = lax.axis_index("s") * NC + lax.axis_index("c")
    in_bufs = (iv0, iv1)
    out_bufs = (ov0, ov1)
    in_sems = (si0, si1)
    out_sems = (so0, so1)

    def unit_addr(u):
        ug = wid * NUNIT + u
        stripe = ug // G
        g = ug % G
        return g, stripe * W

    def start_in(u, b):
        g, col = unit_addr(u)
        pltpu.make_async_copy(
            in_hbm.at[pl.ds(IR * g, IR), pl.ds(col, W)], in_bufs[b], in_sems[b]
        ).start()

    def wait_in(b):
        pltpu.make_async_copy(
            in_hbm.at[pl.ds(0, IR), pl.ds(0, W)], in_bufs[b], in_sems[b]
        ).wait()

    def start_out(u, b):
        g, col = unit_addr(u)
        pltpu.make_async_copy(
            out_bufs[b], out_hbm.at[pl.ds(OR * g, OR), pl.ds(col, W)], out_sems[b]
        ).start()

    def wait_out(b):
        pltpu.make_async_copy(
            out_bufs[b], out_hbm.at[pl.ds(0, OR), pl.ds(0, W)], out_sems[b]
        ).wait()

    def compute(b):
        in_v = in_bufs[b]
        out_v = out_bufs[b]

        @functools.partial(plsc.parallel_loop, 0, OR, unroll=2)
        def row_body(d):
            for v in range(W // L):
                acc = in_v[d, pl.ds(v * L, L)]
                for j in range(1, S):
                    acc = acc + in_v[j * D + d, pl.ds(v * L, L)]
                out_v[d, pl.ds(v * L, L)] = acc * (1.0 / S)

    start_in(0, 0)
    start_in(1, 1)

    def loop_body(i, carry):
        for b in range(2):
            u = 2 * i + b
            wait_in(b)

            @pl.when(u >= 2)
            def _():
                wait_out(b)

            compute(b)
            start_out(u, b)

            @pl.when(u + 2 < NUNIT)
            def _():
                start_in(u + 2, b)

        return carry

    lax.fori_loop(0, NUNIT // 2, loop_body, 0)
    wait_out(0)
    wait_out(1)


@jax.jit
def kernel(emb_vector):
    x_t = jnp.transpose(emb_vector, (1, 2, 0)).reshape(F * D, B)
    mesh = plsc.VectorSubcoreMesh(core_axis_name="c", subcore_axis_name="s")
    out_t = pl.kernel(
        _body,
        out_type=jax.ShapeDtypeStruct((G * D, B), jnp.float32),
        mesh=mesh,
        compiler_params=pltpu.CompilerParams(use_tc_tiling_on_sc=True),
        scratch_types=[
            pltpu.VMEM((IR, W), jnp.float32),
            pltpu.VMEM((IR, W), jnp.float32),
            pltpu.VMEM((OR, W), jnp.float32),
            pltpu.VMEM((OR, W), jnp.float32),
            pltpu.SemaphoreType.DMA,
            pltpu.SemaphoreType.DMA,
            pltpu.SemaphoreType.DMA,
            pltpu.SemaphoreType.DMA,
        ],
    )(x_t)
    return jnp.transpose(out_t.reshape(G, D, B), (2, 0, 1))
